# Initial kernel scaffold; baseline (speedup 1.0000x reference)
import jax
import jax.numpy as jnp
from jax.experimental import pallas as pl


def kernel(edge_index, edge_attr, W):
    raise NotImplementedError("write your pallas kernel here")


# speedup vs baseline: 6.7392x; 6.7392x over previous
import functools

import jax
import jax.numpy as jnp
from jax import lax
from jax.experimental import pallas as pl
from jax.experimental.pallas import tpu as pltpu
from jax.experimental.pallas import tpu_sc as plsc

N_NODES = 10000
N_PAD = 10240
E = 320000
D = 128

NC = 2
NS = 16
NW = NC * NS
EDGES_PER_W = E // NW
CHUNK = 80
NCHUNK = EDGES_PER_W // CHUNK
ROWS_PER_TILE = N_PAD // NS


@functools.cache
def _sc_scatter():
    return functools.partial(
        pl.kernel,
        mesh=plsc.VectorSubcoreMesh(core_axis_name="c", subcore_axis_name="s"),
        out_type=jax.ShapeDtypeStruct((NC * N_PAD, D), jnp.float32),
        scratch_types=[
            pltpu.VMEM_SHARED((N_PAD, D), jnp.float32),
            pltpu.VMEM((NCHUNK, CHUNK), jnp.int32),
            pltpu.VMEM((CHUNK, D), jnp.float32),
            pltpu.VMEM((CHUNK, D), jnp.float32),
            pltpu.SemaphoreType.DMA,
            pltpu.SemaphoreType.DMA,
        ],
    )(_sc_scatter_body)


def _sc_scatter_body(dst_hbm, attr_hbm, zeros_hbm, out_hbm,
                     agg_sh, idx_v, rows0, rows1, sem0, sem1):
    c = lax.axis_index("c")
    s = lax.axis_index("s")
    w = c * NS + s
    base_edge = w * EDGES_PER_W

    r0 = s * ROWS_PER_TILE
    pltpu.sync_copy(zeros_hbm.at[pl.ds(r0, ROWS_PER_TILE)],
                    agg_sh.at[pl.ds(r0, ROWS_PER_TILE)])

    pltpu.sync_copy(dst_hbm.at[w], idx_v)

    plsc.subcore_barrier()

    rows = (rows0, rows1)
    sems = (sem0, sem1)

    def start(i, b):
        pltpu.make_async_copy(
            attr_hbm.at[pl.ds(base_edge + i * CHUNK, CHUNK)],
            rows[b], sems[b]).start()

    def finish(i, b):
        pltpu.make_async_copy(
            attr_hbm.at[pl.ds(base_edge + i * CHUNK, CHUNK)],
            rows[b], sems[b]).wait()
        pltpu.sync_copy(rows[b], agg_sh.at[idx_v.at[i]], add=True)

    start(0, 0)

    def body(g, _):
        i0 = 2 * g
        start(i0 + 1, 1)
        finish(i0, 0)

        @pl.when(i0 + 2 < NCHUNK)
        def _():
            start(i0 + 2, 0)
        finish(i0 + 1, 1)
        return 0

    lax.fori_loop(0, NCHUNK // 2, body, 0)
    finish(NCHUNK - 1, 0)

    plsc.subcore_barrier()

    pltpu.sync_copy(agg_sh.at[pl.ds(r0, ROWS_PER_TILE)],
                    out_hbm.at[pl.ds(c * N_PAD + r0, ROWS_PER_TILE)])


_BR = 2048


def _tc_linear_body(p_ref, wt_ref, o_ref):
    s = p_ref[0] + p_ref[1]
    o_ref[...] = jnp.dot(s, wt_ref[...], preferred_element_type=jnp.float32)


_tc_linear = pl.pallas_call(
    _tc_linear_body,
    grid=(N_PAD // _BR,),
    in_specs=[
        pl.BlockSpec((NC, _BR, D), lambda i: (0, i, 0)),
        pl.BlockSpec((D, D), lambda i: (0, 0)),
    ],
    out_specs=pl.BlockSpec((_BR, D), lambda i: (i, 0)),
    out_shape=jax.ShapeDtypeStruct((N_PAD, D), jnp.float32),
)


@jax.jit
def kernel(edge_index, edge_attr, W):
    dst = edge_index[1].reshape(NW, NCHUNK, CHUNK)
    zeros = jnp.zeros((N_PAD, D), jnp.float32)
    partials = _sc_scatter()(dst, edge_attr, zeros)
    out = _tc_linear(partials.reshape(NC, N_PAD, D), W.T)
    return out[:N_NODES]


# speedup vs baseline: 7.0580x; 1.0473x over previous
import functools

import jax
import jax.numpy as jnp
from jax import lax
from jax.experimental import pallas as pl
from jax.experimental.pallas import tpu as pltpu
from jax.experimental.pallas import tpu_sc as plsc

N_NODES = 10000
N_PAD = 10240
E = 320000
D = 128

NC = 2
NS = 16
NW = NC * NS
EDGES_PER_W = E // NW
CHUNK = 80
NCHUNK = EDGES_PER_W // CHUNK
ROWS_PER_TILE = N_PAD // NS


@functools.cache
def _sc_scatter():
    return functools.partial(
        pl.kernel,
        mesh=plsc.VectorSubcoreMesh(core_axis_name="c", subcore_axis_name="s"),
        out_type=jax.ShapeDtypeStruct((NC * N_NODES, D), jnp.float32),
        scratch_types=[
            pltpu.VMEM_SHARED((N_PAD, D), jnp.float32),
            pltpu.VMEM((NCHUNK, CHUNK), jnp.int32),
            pltpu.VMEM((CHUNK, D), jnp.float32),
            pltpu.VMEM((CHUNK, D), jnp.float32),
            pltpu.SemaphoreType.DMA,
            pltpu.SemaphoreType.DMA,
        ],
    )(_sc_scatter_body)


def _sc_scatter_body(dst_hbm, attr_hbm, out_hbm,
                     agg_sh, idx_v, rows0, rows1, sem0, sem1):
    c = lax.axis_index("c")
    s = lax.axis_index("s")
    w = c * NS + s
    base_edge = w * EDGES_PER_W

    r0 = s * ROWS_PER_TILE
    zero16 = jnp.zeros((16,), jnp.float32)

    def zbody(t, _):
        rows0[t // (D // 16), pl.ds((t % (D // 16)) * 16, 16)] = zero16
        return 0

    lax.fori_loop(0, CHUNK * (D // 16), zbody, 0)
    for k in range(ROWS_PER_TILE // CHUNK):
        pltpu.sync_copy(rows0, agg_sh.at[pl.ds(r0 + k * CHUNK, CHUNK)])

    pltpu.sync_copy(dst_hbm.at[w], idx_v)

    plsc.subcore_barrier()

    rows = (rows0, rows1)
    sems = (sem0, sem1)

    def start(i, b):
        pltpu.make_async_copy(
            attr_hbm.at[pl.ds(base_edge + i * CHUNK, CHUNK)],
            rows[b], sems[b]).start()

    def finish(i, b):
        pltpu.make_async_copy(
            attr_hbm.at[pl.ds(base_edge + i * CHUNK, CHUNK)],
            rows[b], sems[b]).wait()
        pltpu.sync_copy(rows[b], agg_sh.at[idx_v.at[i]], add=True)

    start(0, 0)

    def body(g, _):
        i0 = 2 * g
        start(i0 + 1, 1)
        finish(i0, 0)

        @pl.when(i0 + 2 < NCHUNK)
        def _():
            start(i0 + 2, 0)
        finish(i0 + 1, 1)
        return 0

    lax.fori_loop(0, NCHUNK // 2, body, 0)
    finish(NCHUNK - 1, 0)

    plsc.subcore_barrier()

    tail = N_NODES - (NS - 1) * ROWS_PER_TILE

    @pl.when(s == NS - 1)
    def _():
        pltpu.sync_copy(agg_sh.at[pl.ds(r0, tail)],
                        out_hbm.at[pl.ds(c * N_NODES + r0, tail)])

    @pl.when(s != NS - 1)
    def _():
        pltpu.sync_copy(agg_sh.at[pl.ds(r0, ROWS_PER_TILE)],
                        out_hbm.at[pl.ds(c * N_NODES + r0, ROWS_PER_TILE)])


_BR = 2000


def _tc_linear_body(p_ref, wt_ref, o_ref):
    s = p_ref[0] + p_ref[1]
    o_ref[...] = jnp.dot(s, wt_ref[...], preferred_element_type=jnp.float32)


_tc_linear = pl.pallas_call(
    _tc_linear_body,
    grid=(N_NODES // _BR,),
    in_specs=[
        pl.BlockSpec((NC, _BR, D), lambda i: (0, i, 0)),
        pl.BlockSpec((D, D), lambda i: (0, 0)),
    ],
    out_specs=pl.BlockSpec((_BR, D), lambda i: (i, 0)),
    out_shape=jax.ShapeDtypeStruct((N_NODES, D), jnp.float32),
)


@jax.jit
def kernel(edge_index, edge_attr, W):
    dst = edge_index[1].reshape(NW, NCHUNK, CHUNK)
    partials = _sc_scatter()(dst, edge_attr)
    return _tc_linear(partials.reshape(NC, N_NODES, D), W.T)
